# baseline (speedup 1.0000x reference)
import jax
import jax.numpy as jnp
from jax import lax
from jax.experimental import pallas as pl
from jax.experimental.pallas import tpu as pltpu
from jax.experimental.pallas import tpu_sc as plsc

_N = 10000
_E = 320000
_H = 128

_NC = 2
_NS = 16
_NW = _NC * _NS
_EB = 128
_BPT = 80
_EP = _NW * _BPT * _EB
_ER = _EP // _EB
_NP = _N + 8
_ZR = 200

_RB = 2000
_EBT = 4000

_mesh = plsc.VectorSubcoreMesh(core_axis_name="c", subcore_axis_name="s")


def _deg_body(dst_hbm, out_hbm, didx, ones_v, stage_d, acc_s):
    c = lax.axis_index("c")
    s = lax.axis_index("s")
    w = s * _NC + c
    pltpu.sync_copy(dst_hbm.at[pl.ds(w * _BPT, _BPT)], didx)
    zero16 = jnp.zeros((16,), jnp.float32)
    one16 = jnp.ones((16,), jnp.float32)
    for i in range(_EB // 16):
        ones_v[pl.ds(i * 16, 16)] = one16
    @pl.when(s < 5)
    def _z():
        @pl.loop(0, 2000 // 16)
        def _f(i):
            stage_d[pl.ds(i * 16, 16)] = zero16
        pltpu.sync_copy(stage_d, acc_s.at[pl.ds(s * 2000, 2000)])

    plsc.subcore_barrier()

    @pl.loop(0, _BPT)
    def _blk(b):
        pltpu.sync_copy(ones_v, acc_s.at[didx.at[b]], add=True)

    plsc.subcore_barrier()

    @pl.when(s < 5)
    def _w():
        pltpu.sync_copy(acc_s.at[pl.ds(s * 2000, 2000)], stage_d)
        pltpu.sync_copy(stage_d, out_hbm.at[pl.ds(c * _N + s * 2000, 2000)])


_deg_hist = pl.kernel(
    _deg_body,
    out_type=jax.ShapeDtypeStruct((_NC * _N,), jnp.float32),
    mesh=_mesh,
    scratch_types=[
        pltpu.VMEM((_BPT, _EB), jnp.int32),
        pltpu.VMEM((_EB,), jnp.float32),
        pltpu.VMEM((2000,), jnp.float32),
        pltpu.VMEM_SHARED((_NP,), jnp.float32),
    ],
)


_IG = 16
_NG = _BPT // _IG
_WC = 80


def _segsum_body(g_hbm, src_hbm, dst_hbm, zeros_hbm, out_hbm,
                 sidx, didx, r0, r1, acc_s,
                 gm0, gm1, sm0, sm1, im0, im1):
    c = lax.axis_index("c")
    s = lax.axis_index("s")
    w = s * _NC + c
    rows = [r0, r1]
    gsem = [gm0, gm1]
    ssem = [sm0, sm1]
    isem = [im0, im1]
    rb = w * _BPT

    def ifetch_start(g):
        p = g % 2
        pltpu.make_async_copy(src_hbm.at[pl.ds(rb + g * _IG, _IG)],
                              sidx.at[p], isem[p]).start()
        pltpu.make_async_copy(dst_hbm.at[pl.ds(rb + g * _IG, _IG)],
                              didx.at[p], isem[p]).start()

    def ifetch_wait(g):
        p = g % 2
        pltpu.make_async_copy(src_hbm.at[pl.ds(rb, _IG)],
                              sidx.at[p], isem[p]).wait()
        pltpu.make_async_copy(dst_hbm.at[pl.ds(rb, _IG)],
                              didx.at[p], isem[p]).wait()

    def gst(g, k, j):
        pltpu.make_async_copy(g_hbm.at[sidx.at[g % 2, k]],
                              rows[j], gsem[j]).start()

    def gwt(j):
        pltpu.make_async_copy(g_hbm.at[sidx.at[0, 0]],
                              rows[j], gsem[j]).wait()

    def sst(g, k, j):
        pltpu.make_async_copy(rows[j], acc_s.at[didx.at[g % 2, k]],
                              ssem[j]).start(add=True)

    def swt(j):
        pltpu.make_async_copy(rows[j], acc_s.at[didx.at[0, 0]],
                              ssem[j]).wait()

    ifetch_start(0)
    stage = r0.at[pl.ds(0, _WC)]
    pltpu.sync_copy(zeros_hbm.at[pl.ds(0, _WC)], stage)
    for t in range(8):
        cid = s + _NS * t
        if t < 7:
            pltpu.sync_copy(stage, acc_s.at[pl.ds(cid * _WC, _WC)])
        else:
            @pl.when(cid < _N // _WC)
            def _zz():
                pltpu.sync_copy(stage, acc_s.at[pl.ds(cid * _WC, _WC)])
    ifetch_wait(0)
    plsc.subcore_barrier()

    gst(0, 0, 0)
    for b in range(_BPT):
        g, k = divmod(b, _IG)
        j = b % 2
        if k == 2 and g + 1 < _NG:
            ifetch_start(g + 1)
        gwt(j)
        sst(g, k, j)
        if b == 0:
            gst(0, 1, 1)
        else:
            swt(1 - j)
            nb = b + 1
            if nb < _BPT:
                ng, nk = divmod(nb, _IG)
                if nk == 0:
                    ifetch_wait(ng)
                gst(ng, nk, 1 - j)
    swt(1)

    plsc.subcore_barrier()

    for t in range(8):
        cid = s + _NS * t

        def _wchunk(cid=cid):
            pltpu.sync_copy(acc_s.at[pl.ds(cid * _WC, _WC)], stage)
            pltpu.sync_copy(stage, out_hbm.at[c, pl.ds(cid * _WC, _WC)])

        if t < 7:
            _wchunk()
        else:
            pl.when(cid < _N // _WC)(_wchunk)


_segsum = pl.kernel(
    _segsum_body,
    out_type=jax.ShapeDtypeStruct((_NC, _N, _H), jnp.float32),
    mesh=_mesh,
    scratch_types=[
        pltpu.VMEM((2, _IG, _EB), jnp.int32),
        pltpu.VMEM((2, _IG, _EB), jnp.int32),
        pltpu.VMEM((_EB, _H), jnp.float32),
        pltpu.VMEM((_EB, _H), jnp.float32),
        pltpu.VMEM_SHARED((_NP, _H), jnp.float32),
        pltpu.SemaphoreType.DMA,
        pltpu.SemaphoreType.DMA,
        pltpu.SemaphoreType.DMA,
        pltpu.SemaphoreType.DMA,
        pltpu.SemaphoreType.DMA,
        pltpu.SemaphoreType.DMA,
    ],
)


def _edge_gather_body(a_hbm, b_hbm, src_hbm, dst_hbm, outa_hbm, outb_hbm,
                      sidx, didx, a0, a1, b0r, b1r,
                      ga0, ga1, gb0, gb1, wa0, wa1, wb0, wb1):
    c = lax.axis_index("c")
    s = lax.axis_index("s")
    w = s * _NC + c
    abuf = [a0, a1]
    bbuf = [b0r, b1r]
    gasem = [ga0, ga1]
    gbsem = [gb0, gb1]
    wasem = [wa0, wa1]
    wbsem = [wb0, wb1]
    rb = w * _BPT
    pltpu.sync_copy(src_hbm.at[pl.ds(rb, _BPT)], sidx)
    pltpu.sync_copy(dst_hbm.at[pl.ds(rb, _BPT)], didx)

    def gsta(b, j):
        pltpu.make_async_copy(a_hbm.at[sidx.at[b]], abuf[j], gasem[j]).start()

    def gstb(b, j):
        pltpu.make_async_copy(b_hbm.at[didx.at[b]], bbuf[j], gbsem[j]).start()

    def gwta(j):
        pltpu.make_async_copy(a_hbm.at[sidx.at[0]], abuf[j], gasem[j]).wait()

    def gwtb(j):
        pltpu.make_async_copy(b_hbm.at[didx.at[0]], bbuf[j], gbsem[j]).wait()

    def wsta(b, j):
        pltpu.make_async_copy(
            abuf[j], outa_hbm.at[pl.ds((rb + b) * _EB, _EB)], wasem[j]).start()

    def wstb(b, j):
        pltpu.make_async_copy(
            bbuf[j], outb_hbm.at[pl.ds((rb + b) * _EB, _EB)], wbsem[j]).start()

    def wwta(j):
        pltpu.make_async_copy(
            abuf[j], outa_hbm.at[pl.ds(rb * _EB, _EB)], wasem[j]).wait()

    def wwtb(j):
        pltpu.make_async_copy(
            bbuf[j], outb_hbm.at[pl.ds(rb * _EB, _EB)], wbsem[j]).wait()

    gsta(0, 0)
    gstb(0, 0)
    gwta(0); wsta(0, 0)
    gwtb(0); wstb(0, 0)
    gsta(1, 1); gstb(1, 1)

    @pl.loop(0, (_BPT - 2) // 2)
    def _main(g):
        for k in range(2):
            b = 1 + 2 * g + k
            j = (1 + k) % 2
            gwta(j)
            wsta(b, j)
            gwtb(j)
            wstb(b, j)
            wwta(1 - j)
            wwtb(1 - j)
            gsta(b + 1, 1 - j)
            gstb(b + 1, 1 - j)

    gwta(1); wsta(_BPT - 1, 1)
    gwtb(1); wstb(_BPT - 1, 1)
    wwta(0); wwtb(0); wwta(1); wwtb(1)


_edge_gather = pl.kernel(
    _edge_gather_body,
    out_type=(jax.ShapeDtypeStruct((_EP, _H), jnp.float32),
              jax.ShapeDtypeStruct((_EP, _H), jnp.float32)),
    mesh=_mesh,
    scratch_types=[
        pltpu.VMEM((_BPT, _EB), jnp.int32),
        pltpu.VMEM((_BPT, _EB), jnp.int32),
        pltpu.VMEM((_EB, _H), jnp.float32),
        pltpu.VMEM((_EB, _H), jnp.float32),
        pltpu.VMEM((_EB, _H), jnp.float32),
        pltpu.VMEM((_EB, _H), jnp.float32),
        pltpu.SemaphoreType.DMA,
        pltpu.SemaphoreType.DMA,
        pltpu.SemaphoreType.DMA,
        pltpu.SemaphoreType.DMA,
        pltpu.SemaphoreType.DMA,
        pltpu.SemaphoreType.DMA,
        pltpu.SemaphoreType.DMA,
        pltpu.SemaphoreType.DMA,
    ],
)


def _enc_kernel(nf, degt, w1, b1, w2, b2, cw, x_out, g_out, dinv_out):
    deg = jnp.sum(degt[...], axis=1, keepdims=True) + 1.0
    dinv = lax.rsqrt(deg)
    x = jnp.maximum(jnp.dot(nf[...], w1[...],
                            preferred_element_type=jnp.float32) + b1[...], 0.0)
    x = jnp.dot(x, w2[...], preferred_element_type=jnp.float32) + b2[...]
    x_out[...] = x
    dinv_out[...] = dinv
    g_out[...] = jnp.dot(x, cw[...], preferred_element_type=jnp.float32) * dinv


def _mid_kernel(parts, g_prev, dinv, bias, w_next, g_out):
    x = (parts[0] + parts[1] + g_prev[...]) * dinv[...] + bias[...]
    x = jnp.maximum(x, 0.0)
    g_out[...] = jnp.dot(x, w_next[...],
                         preferred_element_type=jnp.float32) * dinv[...]


def _last_kernel(parts, g_prev, dinv, bias, w_top, b_top, w_bot, a_out, b_out):
    x = (parts[0] + parts[1] + g_prev[...]) * dinv[...] + bias[...]
    a_out[...] = jnp.dot(x, w_top[...],
                         preferred_element_type=jnp.float32) + b_top[...]
    b_out[...] = jnp.dot(x, w_bot[...],
                         preferred_element_type=jnp.float32)


def _score_kernel(ga, gb, w2, b2, s_out):
    h = jnp.maximum(ga[...] + gb[...], 0.0)
    s = jnp.dot(h, w2[...], preferred_element_type=jnp.float32) + b2[...]
    s_out[...] = jax.nn.sigmoid(s)


def _full(shape):
    return pl.BlockSpec(shape, lambda i: (0,) * len(shape))


def _rows(shape):
    return pl.BlockSpec(shape, lambda i: (i,) + (0,) * (len(shape) - 1))


_GRID_N = _N // _RB
_GRID_E = _E // _EBT

_enc_call = pl.pallas_call(
    _enc_kernel,
    grid=(_GRID_N,),
    in_specs=[
        _rows((_RB, _H)), _rows((_RB, _NC)),
        _full((_H, _H)), _full((1, _H)), _full((_H, _H)), _full((1, _H)),
        _full((_H, _H)),
    ],
    out_specs=[_rows((_RB, _H)), _rows((_RB, _H)), _rows((_RB, 1))],
    out_shape=[
        jax.ShapeDtypeStruct((_N, _H), jnp.float32),
        jax.ShapeDtypeStruct((_N, _H), jnp.float32),
        jax.ShapeDtypeStruct((_N, 1), jnp.float32),
    ],
)

_mid_call = pl.pallas_call(
    _mid_kernel,
    grid=(_GRID_N,),
    in_specs=[
        pl.BlockSpec((_NC, _RB, _H), lambda i: (0, i, 0)),
        _rows((_RB, _H)), _rows((_RB, 1)), _full((1, _H)), _full((_H, _H)),
    ],
    out_specs=[_rows((_RB, _H))],
    out_shape=[jax.ShapeDtypeStruct((_N, _H), jnp.float32)],
)

_last_call = pl.pallas_call(
    _last_kernel,
    grid=(_GRID_N,),
    in_specs=[
        pl.BlockSpec((_NC, _RB, _H), lambda i: (0, i, 0)),
        _rows((_RB, _H)), _rows((_RB, 1)), _full((1, _H)),
        _full((_H, _H)), _full((1, _H)), _full((_H, _H)),
    ],
    out_specs=[_rows((_RB, _H)), _rows((_RB, _H))],
    out_shape=[
        jax.ShapeDtypeStruct((_N, _H), jnp.float32),
        jax.ShapeDtypeStruct((_N, _H), jnp.float32),
    ],
)

_score_call = pl.pallas_call(
    _score_kernel,
    grid=(_GRID_E,),
    in_specs=[
        _rows((_EBT, _H)), _rows((_EBT, _H)),
        _full((_H, 1)), _full((1, 1)),
    ],
    out_specs=[_rows((_EBT, 1))],
    out_shape=[jax.ShapeDtypeStruct((_E, 1), jnp.float32)],
)


def kernel(node_features, edge_index, enc_w1, enc_b1, enc_w2, enc_b2,
           conv1_w, conv1_b, conv2_w, conv2_b, conv3_w, conv3_b,
           cls_w1, cls_b1, cls_w2, cls_b2):
    src = edge_index[0]
    dst = edge_index[1]
    pad = _EP - _E
    src2d = jnp.concatenate(
        [src, jnp.zeros((pad,), jnp.int32)]).reshape(_ER, _EB)
    dst2d = jnp.concatenate(
        [dst, jnp.full((pad,), _N, jnp.int32)]).reshape(_ER, _EB)
    zeros = jnp.zeros((_N, _H), jnp.float32)

    deg_parts = _deg_hist(dst2d)
    degt = deg_parts.reshape(_NC, _N).T

    x, g1, dinv = _enc_call(
        node_features, degt, enc_w1, enc_b1.reshape(1, _H),
        enc_w2, enc_b2.reshape(1, _H), conv1_w)

    p1 = _segsum(g1, src2d, dst2d, zeros)
    (g2,) = _mid_call(p1, g1, dinv, conv1_b.reshape(1, _H), conv2_w)

    p2 = _segsum(g2, src2d, dst2d, zeros)
    (g3,) = _mid_call(p2, g2, dinv, conv2_b.reshape(1, _H), conv3_w)

    p3 = _segsum(g3, src2d, dst2d, zeros)
    a_nodes, b_nodes = _last_call(
        p3, g3, dinv, conv3_b.reshape(1, _H),
        cls_w1[:_H], cls_b1.reshape(1, _H), cls_w1[_H:])

    ga, gb = _edge_gather(a_nodes, b_nodes, src2d, dst2d)
    (scores,) = _score_call(ga, gb, cls_w2, cls_b2.reshape(1, 1))
    return scores.reshape(_E)


# speedup vs baseline: 1.0421x; 1.0421x over previous
import jax
import jax.numpy as jnp
from jax import lax
from jax.experimental import pallas as pl
from jax.experimental.pallas import tpu as pltpu
from jax.experimental.pallas import tpu_sc as plsc

_N = 10000
_E = 320000
_H = 128

_NC = 2
_NS = 16
_NW = _NC * _NS
_EB = 64
_BPT = 160
_EP = _NW * _BPT * _EB
_ER = _EP // _EB
_NP = _N + 8
_ZR = 200

_RB = 2000
_EBT = 4000

_mesh = plsc.VectorSubcoreMesh(core_axis_name="c", subcore_axis_name="s")


def _deg_body(dst_hbm, out_hbm, didx, ones_v, stage_d, acc_s):
    c = lax.axis_index("c")
    s = lax.axis_index("s")
    w = s * _NC + c
    pltpu.sync_copy(dst_hbm.at[pl.ds(w * _BPT, _BPT)], didx)
    zero16 = jnp.zeros((16,), jnp.float32)
    one16 = jnp.ones((16,), jnp.float32)
    for i in range(_EB // 16):
        ones_v[pl.ds(i * 16, 16)] = one16
    @pl.when(s < 5)
    def _z():
        @pl.loop(0, 2000 // 16)
        def _f(i):
            stage_d[pl.ds(i * 16, 16)] = zero16
        pltpu.sync_copy(stage_d, acc_s.at[pl.ds(s * 2000, 2000)])

    plsc.subcore_barrier()

    @pl.loop(0, _BPT)
    def _blk(b):
        pltpu.sync_copy(ones_v, acc_s.at[didx.at[b]], add=True)

    plsc.subcore_barrier()

    @pl.when(s < 5)
    def _w():
        pltpu.sync_copy(acc_s.at[pl.ds(s * 2000, 2000)], stage_d)
        pltpu.sync_copy(stage_d, out_hbm.at[pl.ds(c * _N + s * 2000, 2000)])


_deg_hist = pl.kernel(
    _deg_body,
    out_type=jax.ShapeDtypeStruct((_NC * _N,), jnp.float32),
    mesh=_mesh,
    scratch_types=[
        pltpu.VMEM((_BPT, _EB), jnp.int32),
        pltpu.VMEM((_EB,), jnp.float32),
        pltpu.VMEM((2000,), jnp.float32),
        pltpu.VMEM_SHARED((_NP,), jnp.float32),
    ],
)


_IG = 16
_NG = _BPT // _IG
_WC = 40


def _segsum_body(g_hbm, src_hbm, dst_hbm, zeros_hbm, out_hbm,
                 sidx, didx, r0, r1, r2, r3, acc_s,
                 gm0, gm1, gm2, gm3, sm0, sm1, sm2, sm3, isem):
    c = lax.axis_index("c")
    s = lax.axis_index("s")
    w = s * _NC + c
    rows = [r0, r1, r2, r3]
    gsem = [gm0, gm1, gm2, gm3]
    ssem = [sm0, sm1, sm2, sm3]
    rb = w * _BPT

    def ifetch_start(g):
        p = lax.rem(g, 2)
        pltpu.make_async_copy(src_hbm.at[pl.ds(rb + g * _IG, _IG)],
                              sidx.at[p], isem.at[p]).start()
        pltpu.make_async_copy(dst_hbm.at[pl.ds(rb + g * _IG, _IG)],
                              didx.at[p], isem.at[p]).start()

    def ifetch_wait(g):
        p = lax.rem(g, 2)
        pltpu.make_async_copy(src_hbm.at[pl.ds(rb, _IG)],
                              sidx.at[p], isem.at[p]).wait()
        pltpu.make_async_copy(dst_hbm.at[pl.ds(rb, _IG)],
                              didx.at[p], isem.at[p]).wait()

    def gst(b, j):
        g = lax.div(b, _IG)
        pltpu.make_async_copy(g_hbm.at[sidx.at[lax.rem(g, 2), lax.rem(b, _IG)]],
                              rows[j], gsem[j]).start()

    def gwt(j):
        pltpu.make_async_copy(g_hbm.at[sidx.at[0, 0]],
                              rows[j], gsem[j]).wait()

    def sst(b, j):
        g = lax.div(b, _IG)
        pltpu.make_async_copy(
            rows[j], acc_s.at[didx.at[lax.rem(g, 2), lax.rem(b, _IG)]],
            ssem[j]).start(add=True)

    def swt(j):
        pltpu.make_async_copy(rows[j], acc_s.at[didx.at[0, 0]],
                              ssem[j]).wait()

    ifetch_start(0)
    stage = r0.at[pl.ds(0, _WC)]
    pltpu.sync_copy(zeros_hbm.at[pl.ds(0, _WC)], stage)

    @pl.loop(0, 16)
    def _zz(t):
        cid = s + _NS * t

        @pl.when(cid < _N // _WC)
        def _zc():
            pltpu.sync_copy(stage, acc_s.at[pl.ds(cid * _WC, _WC)])

    ifetch_wait(0)
    plsc.subcore_barrier()

    def step(b, j, issue_next=True):
        jj = (j + 2) % 4
        gwt(j)
        sst(b, j)
        swt(jj)
        if issue_next:
            nb = b + 2

            @pl.when(lax.rem(nb, _IG) == 0)
            def _ifw():
                ifetch_wait(lax.div(nb, _IG))

            gst(nb, jj)

        @pl.when(lax.rem(b, _IG) == 2)
        def _ifs():
            g1 = lax.div(b, _IG) + 1

            @pl.when(g1 < _NG)
            def _ifs2():
                ifetch_start(g1)

    gst(0, 0)
    gst(1, 1)
    gwt(0); sst(0, 0); gst(2, 2)
    gwt(1); sst(1, 1); gst(3, 3)

    @pl.loop(0, (_BPT - 4) // 4)
    def _main(gg):
        b = 2 + 4 * gg
        step(b, 2)
        step(b + 1, 3)
        step(b + 2, 0)
        step(b + 3, 1)

    step(_BPT - 2, 2, issue_next=False)
    step(_BPT - 1, 3, issue_next=False)
    swt(2)
    swt(3)

    plsc.subcore_barrier()

    @pl.loop(0, 16)
    def _wo(t):
        cid = s + _NS * t

        @pl.when(cid < _N // _WC)
        def _wc():
            pltpu.sync_copy(acc_s.at[pl.ds(cid * _WC, _WC)], stage)
            pltpu.sync_copy(stage, out_hbm.at[c, pl.ds(cid * _WC, _WC)])


_segsum = pl.kernel(
    _segsum_body,
    out_type=jax.ShapeDtypeStruct((_NC, _N, _H), jnp.float32),
    mesh=_mesh,
    scratch_types=[
        pltpu.VMEM((2, _IG, _EB), jnp.int32),
        pltpu.VMEM((2, _IG, _EB), jnp.int32),
        pltpu.VMEM((_EB, _H), jnp.float32),
        pltpu.VMEM((_EB, _H), jnp.float32),
        pltpu.VMEM((_EB, _H), jnp.float32),
        pltpu.VMEM((_EB, _H), jnp.float32),
        pltpu.VMEM_SHARED((_NP, _H), jnp.float32),
        pltpu.SemaphoreType.DMA,
        pltpu.SemaphoreType.DMA,
        pltpu.SemaphoreType.DMA,
        pltpu.SemaphoreType.DMA,
        pltpu.SemaphoreType.DMA,
        pltpu.SemaphoreType.DMA,
        pltpu.SemaphoreType.DMA,
        pltpu.SemaphoreType.DMA,
        pltpu.SemaphoreType.DMA((2,)),
    ],
)


def _edge_gather_body(a_hbm, b_hbm, src_hbm, dst_hbm, outa_hbm, outb_hbm,
                      sidx, didx, a0, a1, a2, a3, b0r, b1r, b2r, b3r,
                      ga0, ga1, ga2, ga3, gb0, gb1, gb2, gb3,
                      wa0, wa1, wa2, wa3, wb0, wb1, wb2, wb3):
    c = lax.axis_index("c")
    s = lax.axis_index("s")
    w = s * _NC + c
    abuf = [a0, a1, a2, a3]
    bbuf = [b0r, b1r, b2r, b3r]
    gasem = [ga0, ga1, ga2, ga3]
    gbsem = [gb0, gb1, gb2, gb3]
    wasem = [wa0, wa1, wa2, wa3]
    wbsem = [wb0, wb1, wb2, wb3]
    rb = w * _BPT
    pltpu.sync_copy(src_hbm.at[pl.ds(rb, _BPT)], sidx)
    pltpu.sync_copy(dst_hbm.at[pl.ds(rb, _BPT)], didx)

    def gsta(b, j):
        pltpu.make_async_copy(a_hbm.at[sidx.at[b]], abuf[j], gasem[j]).start()

    def gstb(b, j):
        pltpu.make_async_copy(b_hbm.at[didx.at[b]], bbuf[j], gbsem[j]).start()

    def gwta(j):
        pltpu.make_async_copy(a_hbm.at[sidx.at[0]], abuf[j], gasem[j]).wait()

    def gwtb(j):
        pltpu.make_async_copy(b_hbm.at[didx.at[0]], bbuf[j], gbsem[j]).wait()

    def wsta(b, j):
        pltpu.make_async_copy(
            abuf[j], outa_hbm.at[pl.ds((rb + b) * _EB, _EB)], wasem[j]).start()

    def wstb(b, j):
        pltpu.make_async_copy(
            bbuf[j], outb_hbm.at[pl.ds((rb + b) * _EB, _EB)], wbsem[j]).start()

    def wwta(j):
        pltpu.make_async_copy(
            abuf[j], outa_hbm.at[pl.ds(rb * _EB, _EB)], wasem[j]).wait()

    def wwtb(j):
        pltpu.make_async_copy(
            bbuf[j], outb_hbm.at[pl.ds(rb * _EB, _EB)], wbsem[j]).wait()

    def estep(b, j, issue_next=True):
        jj = (j + 2) % 4
        gwta(j)
        wsta(b, j)
        gwtb(j)
        wstb(b, j)
        wwta(jj)
        wwtb(jj)
        if issue_next:
            gsta(b + 2, jj)
            gstb(b + 2, jj)

    gsta(0, 0); gstb(0, 0)
    gsta(1, 1); gstb(1, 1)
    gwta(0); wsta(0, 0); gwtb(0); wstb(0, 0); gsta(2, 2); gstb(2, 2)
    gwta(1); wsta(1, 1); gwtb(1); wstb(1, 1); gsta(3, 3); gstb(3, 3)

    @pl.loop(0, (_BPT - 4) // 4)
    def _main(g):
        b = 2 + 4 * g
        estep(b, 2)
        estep(b + 1, 3)
        estep(b + 2, 0)
        estep(b + 3, 1)

    estep(_BPT - 2, 2, issue_next=False)
    estep(_BPT - 1, 3, issue_next=False)
    wwta(2); wwtb(2); wwta(3); wwtb(3)


_edge_gather = pl.kernel(
    _edge_gather_body,
    out_type=(jax.ShapeDtypeStruct((_EP, _H), jnp.float32),
              jax.ShapeDtypeStruct((_EP, _H), jnp.float32)),
    mesh=_mesh,
    scratch_types=(
        [pltpu.VMEM((_BPT, _EB), jnp.int32)] * 2
        + [pltpu.VMEM((_EB, _H), jnp.float32)] * 8
        + [pltpu.SemaphoreType.DMA] * 16
    ),
)


def _enc_kernel(nf, degt, w1, b1, w2, b2, cw, x_out, g_out, dinv_out):
    deg = jnp.sum(degt[...], axis=1, keepdims=True) + 1.0
    dinv = lax.rsqrt(deg)
    x = jnp.maximum(jnp.dot(nf[...], w1[...],
                            preferred_element_type=jnp.float32) + b1[...], 0.0)
    x = jnp.dot(x, w2[...], preferred_element_type=jnp.float32) + b2[...]
    x_out[...] = x
    dinv_out[...] = dinv
    g_out[...] = jnp.dot(x, cw[...], preferred_element_type=jnp.float32) * dinv


def _mid_kernel(parts, g_prev, dinv, bias, w_next, g_out):
    x = (parts[0] + parts[1] + g_prev[...]) * dinv[...] + bias[...]
    x = jnp.maximum(x, 0.0)
    g_out[...] = jnp.dot(x, w_next[...],
                         preferred_element_type=jnp.float32) * dinv[...]


def _last_kernel(parts, g_prev, dinv, bias, w_top, b_top, w_bot, a_out, b_out):
    x = (parts[0] + parts[1] + g_prev[...]) * dinv[...] + bias[...]
    a_out[...] = jnp.dot(x, w_top[...],
                         preferred_element_type=jnp.float32) + b_top[...]
    b_out[...] = jnp.dot(x, w_bot[...],
                         preferred_element_type=jnp.float32)


def _score_kernel(ga, gb, w2, b2, s_out):
    h = jnp.maximum(ga[...] + gb[...], 0.0)
    s = jnp.dot(h, w2[...], preferred_element_type=jnp.float32) + b2[...]
    s_out[...] = jax.nn.sigmoid(s)


def _full(shape):
    return pl.BlockSpec(shape, lambda i: (0,) * len(shape))


def _rows(shape):
    return pl.BlockSpec(shape, lambda i: (i,) + (0,) * (len(shape) - 1))


_GRID_N = _N // _RB
_GRID_E = _E // _EBT

_enc_call = pl.pallas_call(
    _enc_kernel,
    grid=(_GRID_N,),
    in_specs=[
        _rows((_RB, _H)), _rows((_RB, _NC)),
        _full((_H, _H)), _full((1, _H)), _full((_H, _H)), _full((1, _H)),
        _full((_H, _H)),
    ],
    out_specs=[_rows((_RB, _H)), _rows((_RB, _H)), _rows((_RB, 1))],
    out_shape=[
        jax.ShapeDtypeStruct((_N, _H), jnp.float32),
        jax.ShapeDtypeStruct((_N, _H), jnp.float32),
        jax.ShapeDtypeStruct((_N, 1), jnp.float32),
    ],
)

_mid_call = pl.pallas_call(
    _mid_kernel,
    grid=(_GRID_N,),
    in_specs=[
        pl.BlockSpec((_NC, _RB, _H), lambda i: (0, i, 0)),
        _rows((_RB, _H)), _rows((_RB, 1)), _full((1, _H)), _full((_H, _H)),
    ],
    out_specs=[_rows((_RB, _H))],
    out_shape=[jax.ShapeDtypeStruct((_N, _H), jnp.float32)],
)

_last_call = pl.pallas_call(
    _last_kernel,
    grid=(_GRID_N,),
    in_specs=[
        pl.BlockSpec((_NC, _RB, _H), lambda i: (0, i, 0)),
        _rows((_RB, _H)), _rows((_RB, 1)), _full((1, _H)),
        _full((_H, _H)), _full((1, _H)), _full((_H, _H)),
    ],
    out_specs=[_rows((_RB, _H)), _rows((_RB, _H))],
    out_shape=[
        jax.ShapeDtypeStruct((_N, _H), jnp.float32),
        jax.ShapeDtypeStruct((_N, _H), jnp.float32),
    ],
)

_score_call = pl.pallas_call(
    _score_kernel,
    grid=(_GRID_E,),
    in_specs=[
        _rows((_EBT, _H)), _rows((_EBT, _H)),
        _full((_H, 1)), _full((1, 1)),
    ],
    out_specs=[_rows((_EBT, 1))],
    out_shape=[jax.ShapeDtypeStruct((_E, 1), jnp.float32)],
)


def kernel(node_features, edge_index, enc_w1, enc_b1, enc_w2, enc_b2,
           conv1_w, conv1_b, conv2_w, conv2_b, conv3_w, conv3_b,
           cls_w1, cls_b1, cls_w2, cls_b2):
    src = edge_index[0]
    dst = edge_index[1]
    pad = _EP - _E
    src2d = jnp.concatenate(
        [src, jnp.zeros((pad,), jnp.int32)]).reshape(_ER, _EB)
    dst2d = jnp.concatenate(
        [dst, jnp.full((pad,), _N, jnp.int32)]).reshape(_ER, _EB)
    zeros = jnp.zeros((_N, _H), jnp.float32)

    deg_parts = _deg_hist(dst2d)
    degt = deg_parts.reshape(_NC, _N).T

    x, g1, dinv = _enc_call(
        node_features, degt, enc_w1, enc_b1.reshape(1, _H),
        enc_w2, enc_b2.reshape(1, _H), conv1_w)

    p1 = _segsum(g1, src2d, dst2d, zeros)
    (g2,) = _mid_call(p1, g1, dinv, conv1_b.reshape(1, _H), conv2_w)

    p2 = _segsum(g2, src2d, dst2d, zeros)
    (g3,) = _mid_call(p2, g2, dinv, conv2_b.reshape(1, _H), conv3_w)

    p3 = _segsum(g3, src2d, dst2d, zeros)
    a_nodes, b_nodes = _last_call(
        p3, g3, dinv, conv3_b.reshape(1, _H),
        cls_w1[:_H], cls_b1.reshape(1, _H), cls_w1[_H:])

    ga, gb = _edge_gather(a_nodes, b_nodes, src2d, dst2d)
    (scores,) = _score_call(ga, gb, cls_w2, cls_b2.reshape(1, 1))
    return scores.reshape(_E)


# speedup vs baseline: 1.0459x; 1.0036x over previous
import jax
import jax.numpy as jnp
from jax import lax
from jax.experimental import pallas as pl
from jax.experimental.pallas import tpu as pltpu
from jax.experimental.pallas import tpu_sc as plsc

_N = 10000
_E = 320000
_H = 128

_NC = 2
_NS = 16
_NW = _NC * _NS
_EB = 64
_BPT = 160
_EP = _NW * _BPT * _EB
_ER = _EP // _EB
_NP = _N + 8
_ZR = 200

_RB = 2000
_EBT = 4000

_mesh = plsc.VectorSubcoreMesh(core_axis_name="c", subcore_axis_name="s")


def _deg_body(dst_hbm, out_hbm, didx, ones_v, stage_d, acc_s):
    c = lax.axis_index("c")
    s = lax.axis_index("s")
    w = s * _NC + c
    pltpu.sync_copy(dst_hbm.at[pl.ds(w * _BPT, _BPT)], didx)
    zero16 = jnp.zeros((16,), jnp.float32)
    one16 = jnp.ones((16,), jnp.float32)
    for i in range(_EB // 16):
        ones_v[pl.ds(i * 16, 16)] = one16
    @pl.when(s < 5)
    def _z():
        @pl.loop(0, 2000 // 16)
        def _f(i):
            stage_d[pl.ds(i * 16, 16)] = zero16
        pltpu.sync_copy(stage_d, acc_s.at[pl.ds(s * 2000, 2000)])

    plsc.subcore_barrier()

    @pl.loop(0, _BPT)
    def _blk(b):
        pltpu.sync_copy(ones_v, acc_s.at[didx.at[b]], add=True)

    plsc.subcore_barrier()

    @pl.when(s < 5)
    def _w():
        pltpu.sync_copy(acc_s.at[pl.ds(s * 2000, 2000)], stage_d)
        pltpu.sync_copy(stage_d, out_hbm.at[pl.ds(c * _N + s * 2000, 2000)])


_deg_hist = pl.kernel(
    _deg_body,
    out_type=jax.ShapeDtypeStruct((_NC * _N,), jnp.float32),
    mesh=_mesh,
    scratch_types=[
        pltpu.VMEM((_BPT, _EB), jnp.int32),
        pltpu.VMEM((_EB,), jnp.float32),
        pltpu.VMEM((2000,), jnp.float32),
        pltpu.VMEM_SHARED((_NP,), jnp.float32),
    ],
)


_IG = 16
_NG = _BPT // _IG
_WC = 40


def _segsum_body(g_hbm, src_hbm, dst_hbm, zeros_hbm, out_hbm,
                 sidx, didx, r0, r1, r2, r3, acc_s,
                 gm0, gm1, gm2, gm3, sm0, sm1, sm2, sm3, isem):
    c = lax.axis_index("c")
    s = lax.axis_index("s")
    w = s * _NC + c
    rows = [r0, r1, r2, r3]
    gsem = [gm0, gm1, gm2, gm3]
    ssem = [sm0, sm1, sm2, sm3]
    rb = w * _BPT

    def ifetch_start(g):
        p = lax.rem(g, 2)
        pltpu.make_async_copy(src_hbm.at[pl.ds(rb + g * _IG, _IG)],
                              sidx.at[p], isem.at[p]).start()
        pltpu.make_async_copy(dst_hbm.at[pl.ds(rb + g * _IG, _IG)],
                              didx.at[p], isem.at[p]).start()

    def ifetch_wait(g):
        p = lax.rem(g, 2)
        pltpu.make_async_copy(src_hbm.at[pl.ds(rb, _IG)],
                              sidx.at[p], isem.at[p]).wait()
        pltpu.make_async_copy(dst_hbm.at[pl.ds(rb, _IG)],
                              didx.at[p], isem.at[p]).wait()

    def gst(b, j):
        g = lax.div(b, _IG)
        pltpu.make_async_copy(g_hbm.at[sidx.at[lax.rem(g, 2), lax.rem(b, _IG)]],
                              rows[j], gsem[j]).start()

    def gwt(j):
        pltpu.make_async_copy(g_hbm.at[sidx.at[0, 0]],
                              rows[j], gsem[j]).wait()

    def sst(b, j):
        g = lax.div(b, _IG)
        pltpu.make_async_copy(
            rows[j], acc_s.at[didx.at[lax.rem(g, 2), lax.rem(b, _IG)]],
            ssem[j]).start(add=True)

    def swt(j):
        pltpu.make_async_copy(rows[j], acc_s.at[didx.at[0, 0]],
                              ssem[j]).wait()

    ifetch_start(0)
    stage = r0.at[pl.ds(0, _WC)]
    pltpu.sync_copy(zeros_hbm.at[pl.ds(0, _WC)], stage)

    @pl.loop(0, 16)
    def _zz(t):
        cid = s + _NS * t

        @pl.when(cid < _N // _WC)
        def _zc():
            pltpu.sync_copy(stage, acc_s.at[pl.ds(cid * _WC, _WC)])

    ifetch_wait(0)
    plsc.subcore_barrier()

    def step(b, j, issue_next=True):
        jj = (j + 2) % 4
        gwt(j)
        sst(b, j)
        swt(j)
        if issue_next:
            nb = b + 2

            @pl.when(lax.rem(nb, _IG) == 0)
            def _ifw():
                ifetch_wait(lax.div(nb, _IG))

            gst(nb, jj)

        @pl.when(lax.rem(b, _IG) == 2)
        def _ifs():
            g1 = lax.div(b, _IG) + 1

            @pl.when(g1 < _NG)
            def _ifs2():
                ifetch_start(g1)

    gst(0, 0)
    gst(1, 1)
    gwt(0); sst(0, 0); swt(0); gst(2, 2)
    gwt(1); sst(1, 1); swt(1); gst(3, 3)

    @pl.loop(0, (_BPT - 4) // 4)
    def _main(gg):
        b = 2 + 4 * gg
        step(b, 2)
        step(b + 1, 3)
        step(b + 2, 0)
        step(b + 3, 1)

    step(_BPT - 2, 2, issue_next=False)
    step(_BPT - 1, 3, issue_next=False)

    plsc.subcore_barrier()

    @pl.loop(0, 16)
    def _wo(t):
        cid = s + _NS * t

        @pl.when(cid < _N // _WC)
        def _wc():
            pltpu.sync_copy(acc_s.at[pl.ds(cid * _WC, _WC)], stage)
            pltpu.sync_copy(stage, out_hbm.at[c, pl.ds(cid * _WC, _WC)])


_segsum = pl.kernel(
    _segsum_body,
    out_type=jax.ShapeDtypeStruct((_NC, _N, _H), jnp.float32),
    mesh=_mesh,
    scratch_types=[
        pltpu.VMEM((2, _IG, _EB), jnp.int32),
        pltpu.VMEM((2, _IG, _EB), jnp.int32),
        pltpu.VMEM((_EB, _H), jnp.float32),
        pltpu.VMEM((_EB, _H), jnp.float32),
        pltpu.VMEM((_EB, _H), jnp.float32),
        pltpu.VMEM((_EB, _H), jnp.float32),
        pltpu.VMEM_SHARED((_NP, _H), jnp.float32),
        pltpu.SemaphoreType.DMA,
        pltpu.SemaphoreType.DMA,
        pltpu.SemaphoreType.DMA,
        pltpu.SemaphoreType.DMA,
        pltpu.SemaphoreType.DMA,
        pltpu.SemaphoreType.DMA,
        pltpu.SemaphoreType.DMA,
        pltpu.SemaphoreType.DMA,
        pltpu.SemaphoreType.DMA((2,)),
    ],
)


def _edge_gather_body(a_hbm, b_hbm, src_hbm, dst_hbm, outa_hbm, outb_hbm,
                      sidx, didx, a0, a1, a2, a3, b0r, b1r, b2r, b3r,
                      ga0, ga1, ga2, ga3, gb0, gb1, gb2, gb3,
                      wa0, wa1, wa2, wa3, wb0, wb1, wb2, wb3):
    c = lax.axis_index("c")
    s = lax.axis_index("s")
    w = s * _NC + c
    abuf = [a0, a1, a2, a3]
    bbuf = [b0r, b1r, b2r, b3r]
    gasem = [ga0, ga1, ga2, ga3]
    gbsem = [gb0, gb1, gb2, gb3]
    wasem = [wa0, wa1, wa2, wa3]
    wbsem = [wb0, wb1, wb2, wb3]
    rb = w * _BPT
    pltpu.sync_copy(src_hbm.at[pl.ds(rb, _BPT)], sidx)
    pltpu.sync_copy(dst_hbm.at[pl.ds(rb, _BPT)], didx)

    def gsta(b, j):
        pltpu.make_async_copy(a_hbm.at[sidx.at[b]], abuf[j], gasem[j]).start()

    def gstb(b, j):
        pltpu.make_async_copy(b_hbm.at[didx.at[b]], bbuf[j], gbsem[j]).start()

    def gwta(j):
        pltpu.make_async_copy(a_hbm.at[sidx.at[0]], abuf[j], gasem[j]).wait()

    def gwtb(j):
        pltpu.make_async_copy(b_hbm.at[didx.at[0]], bbuf[j], gbsem[j]).wait()

    def wsta(b, j):
        pltpu.make_async_copy(
            abuf[j], outa_hbm.at[pl.ds((rb + b) * _EB, _EB)], wasem[j]).start()

    def wstb(b, j):
        pltpu.make_async_copy(
            bbuf[j], outb_hbm.at[pl.ds((rb + b) * _EB, _EB)], wbsem[j]).start()

    def wwta(j):
        pltpu.make_async_copy(
            abuf[j], outa_hbm.at[pl.ds(rb * _EB, _EB)], wasem[j]).wait()

    def wwtb(j):
        pltpu.make_async_copy(
            bbuf[j], outb_hbm.at[pl.ds(rb * _EB, _EB)], wbsem[j]).wait()

    def estep(b, j, issue_next=True):
        jj = (j + 2) % 4
        gwta(j)
        wsta(b, j)
        gwtb(j)
        wstb(b, j)
        wwta(jj)
        wwtb(jj)
        if issue_next:
            gsta(b + 2, jj)
            gstb(b + 2, jj)

    gsta(0, 0); gstb(0, 0)
    gsta(1, 1); gstb(1, 1)
    gwta(0); wsta(0, 0); gwtb(0); wstb(0, 0); gsta(2, 2); gstb(2, 2)
    gwta(1); wsta(1, 1); gwtb(1); wstb(1, 1); gsta(3, 3); gstb(3, 3)

    @pl.loop(0, (_BPT - 4) // 4)
    def _main(g):
        b = 2 + 4 * g
        estep(b, 2)
        estep(b + 1, 3)
        estep(b + 2, 0)
        estep(b + 3, 1)

    estep(_BPT - 2, 2, issue_next=False)
    estep(_BPT - 1, 3, issue_next=False)
    wwta(2); wwtb(2); wwta(3); wwtb(3)


_edge_gather = pl.kernel(
    _edge_gather_body,
    out_type=(jax.ShapeDtypeStruct((_EP, _H), jnp.float32),
              jax.ShapeDtypeStruct((_EP, _H), jnp.float32)),
    mesh=_mesh,
    scratch_types=(
        [pltpu.VMEM((_BPT, _EB), jnp.int32)] * 2
        + [pltpu.VMEM((_EB, _H), jnp.float32)] * 8
        + [pltpu.SemaphoreType.DMA] * 16
    ),
)


def _enc_kernel(nf, degt, w1, b1, w2, b2, cw, x_out, g_out, dinv_out):
    deg = jnp.sum(degt[...], axis=1, keepdims=True) + 1.0
    dinv = lax.rsqrt(deg)
    x = jnp.maximum(jnp.dot(nf[...], w1[...],
                            preferred_element_type=jnp.float32) + b1[...], 0.0)
    x = jnp.dot(x, w2[...], preferred_element_type=jnp.float32) + b2[...]
    x_out[...] = x
    dinv_out[...] = dinv
    g_out[...] = jnp.dot(x, cw[...], preferred_element_type=jnp.float32) * dinv


def _mid_kernel(parts, g_prev, dinv, bias, w_next, g_out):
    x = (parts[0] + parts[1] + g_prev[...]) * dinv[...] + bias[...]
    x = jnp.maximum(x, 0.0)
    g_out[...] = jnp.dot(x, w_next[...],
                         preferred_element_type=jnp.float32) * dinv[...]


def _last_kernel(parts, g_prev, dinv, bias, w_top, b_top, w_bot, a_out, b_out):
    x = (parts[0] + parts[1] + g_prev[...]) * dinv[...] + bias[...]
    a_out[...] = jnp.dot(x, w_top[...],
                         preferred_element_type=jnp.float32) + b_top[...]
    b_out[...] = jnp.dot(x, w_bot[...],
                         preferred_element_type=jnp.float32)


def _score_kernel(ga, gb, w2, b2, s_out):
    h = jnp.maximum(ga[...] + gb[...], 0.0)
    s = jnp.dot(h, w2[...], preferred_element_type=jnp.float32) + b2[...]
    s_out[...] = jax.nn.sigmoid(s)


def _full(shape):
    return pl.BlockSpec(shape, lambda i: (0,) * len(shape))


def _rows(shape):
    return pl.BlockSpec(shape, lambda i: (i,) + (0,) * (len(shape) - 1))


_GRID_N = _N // _RB
_GRID_E = _E // _EBT

_enc_call = pl.pallas_call(
    _enc_kernel,
    grid=(_GRID_N,),
    in_specs=[
        _rows((_RB, _H)), _rows((_RB, _NC)),
        _full((_H, _H)), _full((1, _H)), _full((_H, _H)), _full((1, _H)),
        _full((_H, _H)),
    ],
    out_specs=[_rows((_RB, _H)), _rows((_RB, _H)), _rows((_RB, 1))],
    out_shape=[
        jax.ShapeDtypeStruct((_N, _H), jnp.float32),
        jax.ShapeDtypeStruct((_N, _H), jnp.float32),
        jax.ShapeDtypeStruct((_N, 1), jnp.float32),
    ],
)

_mid_call = pl.pallas_call(
    _mid_kernel,
    grid=(_GRID_N,),
    in_specs=[
        pl.BlockSpec((_NC, _RB, _H), lambda i: (0, i, 0)),
        _rows((_RB, _H)), _rows((_RB, 1)), _full((1, _H)), _full((_H, _H)),
    ],
    out_specs=[_rows((_RB, _H))],
    out_shape=[jax.ShapeDtypeStruct((_N, _H), jnp.float32)],
)

_last_call = pl.pallas_call(
    _last_kernel,
    grid=(_GRID_N,),
    in_specs=[
        pl.BlockSpec((_NC, _RB, _H), lambda i: (0, i, 0)),
        _rows((_RB, _H)), _rows((_RB, 1)), _full((1, _H)),
        _full((_H, _H)), _full((1, _H)), _full((_H, _H)),
    ],
    out_specs=[_rows((_RB, _H)), _rows((_RB, _H))],
    out_shape=[
        jax.ShapeDtypeStruct((_N, _H), jnp.float32),
        jax.ShapeDtypeStruct((_N, _H), jnp.float32),
    ],
)

_score_call = pl.pallas_call(
    _score_kernel,
    grid=(_GRID_E,),
    in_specs=[
        _rows((_EBT, _H)), _rows((_EBT, _H)),
        _full((_H, 1)), _full((1, 1)),
    ],
    out_specs=[_rows((_EBT, 1))],
    out_shape=[jax.ShapeDtypeStruct((_E, 1), jnp.float32)],
)


def kernel(node_features, edge_index, enc_w1, enc_b1, enc_w2, enc_b2,
           conv1_w, conv1_b, conv2_w, conv2_b, conv3_w, conv3_b,
           cls_w1, cls_b1, cls_w2, cls_b2):
    src = edge_index[0]
    dst = edge_index[1]
    pad = _EP - _E
    src2d = jnp.concatenate(
        [src, jnp.zeros((pad,), jnp.int32)]).reshape(_ER, _EB)
    dst2d = jnp.concatenate(
        [dst, jnp.full((pad,), _N, jnp.int32)]).reshape(_ER, _EB)
    zeros = jnp.zeros((_N, _H), jnp.float32)

    deg_parts = _deg_hist(dst2d)
    degt = deg_parts.reshape(_NC, _N).T

    x, g1, dinv = _enc_call(
        node_features, degt, enc_w1, enc_b1.reshape(1, _H),
        enc_w2, enc_b2.reshape(1, _H), conv1_w)

    p1 = _segsum(g1, src2d, dst2d, zeros)
    (g2,) = _mid_call(p1, g1, dinv, conv1_b.reshape(1, _H), conv2_w)

    p2 = _segsum(g2, src2d, dst2d, zeros)
    (g3,) = _mid_call(p2, g2, dinv, conv2_b.reshape(1, _H), conv3_w)

    p3 = _segsum(g3, src2d, dst2d, zeros)
    a_nodes, b_nodes = _last_call(
        p3, g3, dinv, conv3_b.reshape(1, _H),
        cls_w1[:_H], cls_b1.reshape(1, _H), cls_w1[_H:])

    ga, gb = _edge_gather(a_nodes, b_nodes, src2d, dst2d)
    (scores,) = _score_call(ga, gb, cls_w2, cls_b2.reshape(1, 1))
    return scores.reshape(_E)


# speedup vs baseline: 1.2186x; 1.1651x over previous
import jax
import jax.numpy as jnp
from jax import lax
from jax.experimental import pallas as pl
from jax.experimental.pallas import tpu as pltpu
from jax.experimental.pallas import tpu_sc as plsc

_N = 10000
_E = 320000
_H = 128

_NC = 2
_NS = 16
_NW = _NC * _NS
_EB = 64
_BPT = 320
_K0 = 240
_K1 = _BPT - _K0
_EP = _NS * _BPT * _EB
_ER = _EP // _EB
_NP = _N + 8
_ZR = 200

_RB = 2000
_EBT = 4000

_mesh = plsc.VectorSubcoreMesh(core_axis_name="c", subcore_axis_name="s")


_BPW = _ER // _NW


def _deg_body(dst_hbm, out_hbm, didx, ones_v, stage_d, acc_s):
    c = lax.axis_index("c")
    s = lax.axis_index("s")
    w = s * _NC + c
    pltpu.sync_copy(dst_hbm.at[pl.ds(w * _BPW, _BPW)], didx)
    zero16 = jnp.zeros((16,), jnp.float32)
    one16 = jnp.ones((16,), jnp.float32)
    for i in range(_EB // 16):
        ones_v[pl.ds(i * 16, 16)] = one16
    @pl.when(s < 5)
    def _z():
        @pl.loop(0, 2000 // 16)
        def _f(i):
            stage_d[pl.ds(i * 16, 16)] = zero16
        pltpu.sync_copy(stage_d, acc_s.at[pl.ds(s * 2000, 2000)])

    plsc.subcore_barrier()

    @pl.loop(0, _BPW)
    def _blk(b):
        pltpu.sync_copy(ones_v, acc_s.at[didx.at[b]], add=True)

    plsc.subcore_barrier()

    @pl.when(s < 5)
    def _w():
        pltpu.sync_copy(acc_s.at[pl.ds(s * 2000, 2000)], stage_d)
        pltpu.sync_copy(stage_d, out_hbm.at[pl.ds(c * _N + s * 2000, 2000)])


_deg_hist = pl.kernel(
    _deg_body,
    out_type=jax.ShapeDtypeStruct((_NC * _N,), jnp.float32),
    mesh=_mesh,
    scratch_types=[
        pltpu.VMEM((_BPW, _EB), jnp.int32),
        pltpu.VMEM((_EB,), jnp.float32),
        pltpu.VMEM((2000,), jnp.float32),
        pltpu.VMEM_SHARED((_NP,), jnp.float32),
    ],
)


_IG = 16
_NG = _BPT // _IG
_WC = 40


def _segsum_body(g_hbm, src_hbm, dst_hbm, zeros_hbm, out_hbm,
                 sidx, didx, r0, r1, r2, r3, acc_s,
                 gm0, gm1, gm2, gm3, sm0, sm1, sm2, sm3, isem):
    c = lax.axis_index("c")
    s = lax.axis_index("s")
    w = s * _NC + c
    rows = [r0, r1, r2, r3]
    gsem = [gm0, gm1, gm2, gm3]
    ssem = [sm0, sm1, sm2, sm3]
    del w
    rb = s * _BPT + c * _K0
    nk = jnp.where(c == 0, _K0, _K1)

    def ifetch_start(g):
        p = lax.rem(g, 2)
        pltpu.make_async_copy(src_hbm.at[pl.ds(rb + g * _IG, _IG)],
                              sidx.at[p], isem.at[p]).start()
        pltpu.make_async_copy(dst_hbm.at[pl.ds(rb + g * _IG, _IG)],
                              didx.at[p], isem.at[p]).start()

    def ifetch_wait(g):
        p = lax.rem(g, 2)
        pltpu.make_async_copy(src_hbm.at[pl.ds(rb, _IG)],
                              sidx.at[p], isem.at[p]).wait()
        pltpu.make_async_copy(dst_hbm.at[pl.ds(rb, _IG)],
                              didx.at[p], isem.at[p]).wait()

    def gst(b, j):
        g = lax.div(b, _IG)
        pltpu.make_async_copy(g_hbm.at[sidx.at[lax.rem(g, 2), lax.rem(b, _IG)]],
                              rows[j], gsem[j]).start()

    def gwt(j):
        pltpu.make_async_copy(g_hbm.at[sidx.at[0, 0]],
                              rows[j], gsem[j]).wait()

    def sst(b, j):
        g = lax.div(b, _IG)
        pltpu.make_async_copy(
            rows[j], acc_s.at[didx.at[lax.rem(g, 2), lax.rem(b, _IG)]],
            ssem[j]).start(add=True)

    def swt(j):
        pltpu.make_async_copy(rows[j], acc_s.at[didx.at[0, 0]],
                              ssem[j]).wait()

    ifetch_start(0)
    stage = r0.at[pl.ds(0, _WC)]
    pltpu.sync_copy(zeros_hbm.at[pl.ds(0, _WC)], stage)

    @pl.loop(0, 16)
    def _zz(t):
        cid = s + _NS * t

        @pl.when(cid < _N // _WC)
        def _zc():
            pltpu.sync_copy(stage, acc_s.at[pl.ds(cid * _WC, _WC)])

    ifetch_wait(0)
    plsc.subcore_barrier()

    def step(b, j, issue_next=True):
        jj = (j + 2) % 4
        gwt(j)
        sst(b, j)
        swt(j)
        if issue_next:
            nb = b + 2

            @pl.when(lax.rem(nb, _IG) == 0)
            def _ifw():
                ifetch_wait(lax.div(nb, _IG))

            gst(nb, jj)

        @pl.when(lax.rem(b, _IG) == 2)
        def _ifs():
            g1 = lax.div(b, _IG) + 1

            @pl.when(g1 * _IG < nk)
            def _ifs2():
                ifetch_start(g1)

    gst(0, 0)
    gst(1, 1)
    gwt(0); sst(0, 0); swt(0); gst(2, 2)
    gwt(1); sst(1, 1); swt(1); gst(3, 3)

    @pl.loop(0, lax.div(nk - 4, 4))
    def _main(gg):
        b = 2 + 4 * gg
        step(b, 2)
        step(b + 1, 3)
        step(b + 2, 0)
        step(b + 3, 1)

    step(nk - 2, 2, issue_next=False)
    step(nk - 1, 3, issue_next=False)

    plsc.subcore_barrier()

    @pl.loop(0, 16)
    def _wo(t):
        cid = s + _NS * t

        @pl.when(cid < _N // _WC)
        def _wc():
            pltpu.sync_copy(acc_s.at[pl.ds(cid * _WC, _WC)], stage)
            pltpu.sync_copy(stage, out_hbm.at[c, pl.ds(cid * _WC, _WC)])


_segsum = pl.kernel(
    _segsum_body,
    out_type=jax.ShapeDtypeStruct((_NC, _N, _H), jnp.float32),
    mesh=_mesh,
    scratch_types=[
        pltpu.VMEM((2, _IG, _EB), jnp.int32),
        pltpu.VMEM((2, _IG, _EB), jnp.int32),
        pltpu.VMEM((_EB, _H), jnp.float32),
        pltpu.VMEM((_EB, _H), jnp.float32),
        pltpu.VMEM((_EB, _H), jnp.float32),
        pltpu.VMEM((_EB, _H), jnp.float32),
        pltpu.VMEM_SHARED((_NP, _H), jnp.float32),
        pltpu.SemaphoreType.DMA,
        pltpu.SemaphoreType.DMA,
        pltpu.SemaphoreType.DMA,
        pltpu.SemaphoreType.DMA,
        pltpu.SemaphoreType.DMA,
        pltpu.SemaphoreType.DMA,
        pltpu.SemaphoreType.DMA,
        pltpu.SemaphoreType.DMA,
        pltpu.SemaphoreType.DMA((2,)),
    ],
)


def _edge_gather_body(a_hbm, b_hbm, src_hbm, dst_hbm, outa_hbm, outb_hbm,
                      sidx, didx, a0, a1, a2, a3, b0r, b1r, b2r, b3r,
                      ga0, ga1, ga2, ga3, gb0, gb1, gb2, gb3,
                      wa0, wa1, wa2, wa3, wb0, wb1, wb2, wb3):
    c = lax.axis_index("c")
    s = lax.axis_index("s")
    w = s * _NC + c
    abuf = [a0, a1, a2, a3]
    bbuf = [b0r, b1r, b2r, b3r]
    gasem = [ga0, ga1, ga2, ga3]
    gbsem = [gb0, gb1, gb2, gb3]
    wasem = [wa0, wa1, wa2, wa3]
    wbsem = [wb0, wb1, wb2, wb3]
    del w
    rb = s * _BPT + c * _K0
    nk = jnp.where(c == 0, _K0, _K1)

    @pl.when(c == 0)
    def _f0():
        pltpu.sync_copy(src_hbm.at[pl.ds(rb, _K0)], sidx.at[pl.ds(0, _K0)])
        pltpu.sync_copy(dst_hbm.at[pl.ds(rb, _K0)], didx.at[pl.ds(0, _K0)])

    @pl.when(c == 1)
    def _f1():
        pltpu.sync_copy(src_hbm.at[pl.ds(rb, _K1)], sidx.at[pl.ds(0, _K1)])
        pltpu.sync_copy(dst_hbm.at[pl.ds(rb, _K1)], didx.at[pl.ds(0, _K1)])

    def gsta(b, j):
        pltpu.make_async_copy(a_hbm.at[sidx.at[b]], abuf[j], gasem[j]).start()

    def gstb(b, j):
        pltpu.make_async_copy(b_hbm.at[didx.at[b]], bbuf[j], gbsem[j]).start()

    def gwta(j):
        pltpu.make_async_copy(a_hbm.at[sidx.at[0]], abuf[j], gasem[j]).wait()

    def gwtb(j):
        pltpu.make_async_copy(b_hbm.at[didx.at[0]], bbuf[j], gbsem[j]).wait()

    def wsta(b, j):
        pltpu.make_async_copy(
            abuf[j], outa_hbm.at[pl.ds((rb + b) * _EB, _EB)], wasem[j]).start()

    def wstb(b, j):
        pltpu.make_async_copy(
            bbuf[j], outb_hbm.at[pl.ds((rb + b) * _EB, _EB)], wbsem[j]).start()

    def wwta(j):
        pltpu.make_async_copy(
            abuf[j], outa_hbm.at[pl.ds(rb * _EB, _EB)], wasem[j]).wait()

    def wwtb(j):
        pltpu.make_async_copy(
            bbuf[j], outb_hbm.at[pl.ds(rb * _EB, _EB)], wbsem[j]).wait()

    def estep(b, j, issue_next=True):
        jj = (j + 2) % 4
        gwta(j)
        wsta(b, j)
        gwtb(j)
        wstb(b, j)
        wwta(jj)
        wwtb(jj)
        if issue_next:
            gsta(b + 2, jj)
            gstb(b + 2, jj)

    gsta(0, 0); gstb(0, 0)
    gsta(1, 1); gstb(1, 1)
    gwta(0); wsta(0, 0); gwtb(0); wstb(0, 0); gsta(2, 2); gstb(2, 2)
    gwta(1); wsta(1, 1); gwtb(1); wstb(1, 1); gsta(3, 3); gstb(3, 3)

    @pl.loop(0, lax.div(nk - 4, 4))
    def _main(g):
        b = 2 + 4 * g
        estep(b, 2)
        estep(b + 1, 3)
        estep(b + 2, 0)
        estep(b + 3, 1)

    estep(nk - 2, 2, issue_next=False)
    estep(nk - 1, 3, issue_next=False)
    wwta(2); wwtb(2); wwta(3); wwtb(3)


_edge_gather = pl.kernel(
    _edge_gather_body,
    out_type=(jax.ShapeDtypeStruct((_EP, _H), jnp.float32),
              jax.ShapeDtypeStruct((_EP, _H), jnp.float32)),
    mesh=_mesh,
    scratch_types=(
        [pltpu.VMEM((_K0, _EB), jnp.int32)] * 2
        + [pltpu.VMEM((_EB, _H), jnp.float32)] * 8
        + [pltpu.SemaphoreType.DMA] * 16
    ),
)


def _enc_kernel(nf, degt, w1, b1, w2, b2, cw, x_out, g_out, dinv_out):
    deg = jnp.sum(degt[...], axis=1, keepdims=True) + 1.0
    dinv = lax.rsqrt(deg)
    x = jnp.maximum(jnp.dot(nf[...], w1[...],
                            preferred_element_type=jnp.float32) + b1[...], 0.0)
    x = jnp.dot(x, w2[...], preferred_element_type=jnp.float32) + b2[...]
    x_out[...] = x
    dinv_out[...] = dinv
    g_out[...] = jnp.dot(x, cw[...], preferred_element_type=jnp.float32) * dinv


def _mid_kernel(parts, g_prev, dinv, bias, w_next, g_out):
    x = (parts[0] + parts[1] + g_prev[...]) * dinv[...] + bias[...]
    x = jnp.maximum(x, 0.0)
    g_out[...] = jnp.dot(x, w_next[...],
                         preferred_element_type=jnp.float32) * dinv[...]


def _last_kernel(parts, g_prev, dinv, bias, w_top, b_top, w_bot, a_out, b_out):
    x = (parts[0] + parts[1] + g_prev[...]) * dinv[...] + bias[...]
    a_out[...] = jnp.dot(x, w_top[...],
                         preferred_element_type=jnp.float32) + b_top[...]
    b_out[...] = jnp.dot(x, w_bot[...],
                         preferred_element_type=jnp.float32)


def _score_kernel(ga, gb, w2, b2, s_out):
    h = jnp.maximum(ga[...] + gb[...], 0.0)
    s = jnp.dot(h, w2[...], preferred_element_type=jnp.float32) + b2[...]
    s_out[...] = jax.nn.sigmoid(s)


def _full(shape):
    return pl.BlockSpec(shape, lambda i: (0,) * len(shape))


def _rows(shape):
    return pl.BlockSpec(shape, lambda i: (i,) + (0,) * (len(shape) - 1))


_GRID_N = _N // _RB
_GRID_E = _E // _EBT

_enc_call = pl.pallas_call(
    _enc_kernel,
    grid=(_GRID_N,),
    in_specs=[
        _rows((_RB, _H)), _rows((_RB, _NC)),
        _full((_H, _H)), _full((1, _H)), _full((_H, _H)), _full((1, _H)),
        _full((_H, _H)),
    ],
    out_specs=[_rows((_RB, _H)), _rows((_RB, _H)), _rows((_RB, 1))],
    out_shape=[
        jax.ShapeDtypeStruct((_N, _H), jnp.float32),
        jax.ShapeDtypeStruct((_N, _H), jnp.float32),
        jax.ShapeDtypeStruct((_N, 1), jnp.float32),
    ],
)

_mid_call = pl.pallas_call(
    _mid_kernel,
    grid=(_GRID_N,),
    in_specs=[
        pl.BlockSpec((_NC, _RB, _H), lambda i: (0, i, 0)),
        _rows((_RB, _H)), _rows((_RB, 1)), _full((1, _H)), _full((_H, _H)),
    ],
    out_specs=[_rows((_RB, _H))],
    out_shape=[jax.ShapeDtypeStruct((_N, _H), jnp.float32)],
)

_last_call = pl.pallas_call(
    _last_kernel,
    grid=(_GRID_N,),
    in_specs=[
        pl.BlockSpec((_NC, _RB, _H), lambda i: (0, i, 0)),
        _rows((_RB, _H)), _rows((_RB, 1)), _full((1, _H)),
        _full((_H, _H)), _full((1, _H)), _full((_H, _H)),
    ],
    out_specs=[_rows((_RB, _H)), _rows((_RB, _H))],
    out_shape=[
        jax.ShapeDtypeStruct((_N, _H), jnp.float32),
        jax.ShapeDtypeStruct((_N, _H), jnp.float32),
    ],
)

_score_call = pl.pallas_call(
    _score_kernel,
    grid=(_GRID_E,),
    in_specs=[
        _rows((_EBT, _H)), _rows((_EBT, _H)),
        _full((_H, 1)), _full((1, 1)),
    ],
    out_specs=[_rows((_EBT, 1))],
    out_shape=[jax.ShapeDtypeStruct((_E, 1), jnp.float32)],
)


def kernel(node_features, edge_index, enc_w1, enc_b1, enc_w2, enc_b2,
           conv1_w, conv1_b, conv2_w, conv2_b, conv3_w, conv3_b,
           cls_w1, cls_b1, cls_w2, cls_b2):
    src = edge_index[0]
    dst = edge_index[1]
    pad = _EP - _E
    src2d = jnp.concatenate(
        [src, jnp.zeros((pad,), jnp.int32)]).reshape(_ER, _EB)
    dst2d = jnp.concatenate(
        [dst, jnp.full((pad,), _N, jnp.int32)]).reshape(_ER, _EB)
    zeros = jnp.zeros((_N, _H), jnp.float32)

    deg_parts = _deg_hist(dst2d)
    degt = deg_parts.reshape(_NC, _N).T

    x, g1, dinv = _enc_call(
        node_features, degt, enc_w1, enc_b1.reshape(1, _H),
        enc_w2, enc_b2.reshape(1, _H), conv1_w)

    p1 = _segsum(g1, src2d, dst2d, zeros)
    (g2,) = _mid_call(p1, g1, dinv, conv1_b.reshape(1, _H), conv2_w)

    p2 = _segsum(g2, src2d, dst2d, zeros)
    (g3,) = _mid_call(p2, g2, dinv, conv2_b.reshape(1, _H), conv3_w)

    p3 = _segsum(g3, src2d, dst2d, zeros)
    a_nodes, b_nodes = _last_call(
        p3, g3, dinv, conv3_b.reshape(1, _H),
        cls_w1[:_H], cls_b1.reshape(1, _H), cls_w1[_H:])

    ga, gb = _edge_gather(a_nodes, b_nodes, src2d, dst2d)
    (scores,) = _score_call(ga, gb, cls_w2, cls_b2.reshape(1, 1))
    return scores.reshape(_E)
